# Initial kernel scaffold; baseline (speedup 1.0000x reference)
import jax
import jax.numpy as jnp
from jax.experimental import pallas as pl


def kernel(point_cloud, W1a, b1a, W1b, b1b, W2a, b2a, W2b, b2b, W3a, b3a, W3b, b3b, Wf1, bf1, gf1, bef1, Wf2, bf2, gf2, bef2):
    raise NotImplementedError("write your pallas kernel here")


# speedup vs baseline: 1.2691x; 1.2691x over previous
import functools

import jax
import jax.numpy as jnp
from jax.experimental import pallas as pl

EPS = 1e-5


def _fps_kernel(pc_ref, fidx_ref, newxyz_ref, *, npoint, n):
    b = pc_ref.shape[0]
    X = pc_ref[:, 0, :]
    Y = pc_ref[:, 1, :]
    Z = pc_ref[:, 2, :]
    lane = jax.lax.broadcasted_iota(jnp.int32, (b, n), 1)
    subl = jax.lax.broadcasted_iota(jnp.int32, (b, n), 0)
    subl1 = jax.lax.broadcasted_iota(jnp.int32, (b, 1), 0)
    sublp = jax.lax.broadcasted_iota(jnp.int32, (b, npoint), 0)
    col = jax.lax.broadcasted_iota(jnp.int32, (b, npoint), 1)

    def body(i, carry):
        dists, far, aidx, ax, ay, az = carry
        sel = col == i
        aidx = jnp.where(sel, jnp.broadcast_to(far, aidx.shape), aidx)
        mask = lane == far
        cx = jnp.sum(jnp.where(mask, X, 0.0), axis=1, keepdims=True)
        cy = jnp.sum(jnp.where(mask, Y, 0.0), axis=1, keepdims=True)
        cz = jnp.sum(jnp.where(mask, Z, 0.0), axis=1, keepdims=True)
        ax = jnp.where(sel, jnp.broadcast_to(cx, ax.shape), ax)
        ay = jnp.where(sel, jnp.broadcast_to(cy, ay.shape), ay)
        az = jnp.where(sel, jnp.broadcast_to(cz, az.shape), az)
        d = (X - cx) ** 2 + (Y - cy) ** 2 + (Z - cz) ** 2
        dists = jnp.minimum(dists, d)
        m = jnp.max(dists, axis=1, keepdims=True)
        far = jnp.min(jnp.where(dists == m, lane, n), axis=1, keepdims=True)
        return dists, far, aidx, ax, ay, az

    dists0 = jnp.maximum(subl.astype(jnp.float32), 1e10)
    far0 = jnp.minimum(subl1, 0)
    aidx0 = jnp.minimum(sublp, 0)
    az0 = aidx0.astype(jnp.float32)
    _, _, aidx, ax, ay, az = jax.lax.fori_loop(
        0, npoint, body, (dists0, far0, aidx0, az0, az0, az0))
    fidx_ref[...] = aidx
    newxyz_ref[:, 0, :] = ax
    newxyz_ref[:, 1, :] = ay
    newxyz_ref[:, 2, :] = az


def _fps(pc, npoint):
    b, _, n = pc.shape
    return pl.pallas_call(
        functools.partial(_fps_kernel, npoint=npoint, n=n),
        out_shape=[jax.ShapeDtypeStruct((b, npoint), jnp.int32),
                   jax.ShapeDtypeStruct((b, 3, npoint), jnp.float32)],
    )(pc)


def _query_knn(k, xyz_t, new_xyz_t):
    sq = (jnp.sum(new_xyz_t ** 2, -1)[:, :, None] + jnp.sum(xyz_t ** 2, -1)[:, None, :]
          - 2.0 * jnp.einsum('bsd,bnd->bsn', new_xyz_t, xyz_t))
    _, idx = jax.lax.top_k(-sq, k)
    return idx


def _group(points, idx):
    B, C, N = points.shape
    S, K = idx.shape[1], idx.shape[2]
    flat = jnp.broadcast_to(idx.reshape(B, 1, S * K), (B, C, S * K))
    return jnp.take_along_axis(points, flat, axis=2).reshape(B, C, S, K)


def _conv_mlp(x, params):
    n = len(params)
    for i, (W, b) in enumerate(params):
        x = jnp.einsum('oc,bcsk->bosk', W, x) + b[None, :, None, None]
        if i < n - 1:
            x = jax.nn.relu(x)
    return x


def _sa_knn(xyz, points, npoint, k, params):
    _, new_xyz = _fps(xyz, npoint)
    xyz_t = jnp.transpose(xyz, (0, 2, 1))
    idx = _query_knn(k, xyz_t, jnp.transpose(new_xyz, (0, 2, 1)))
    g_xyz = _group(xyz, idx) - new_xyz[:, :, :, None]
    g_pts = _group(points, idx)
    new_points = jnp.concatenate([g_xyz, g_pts], axis=1)
    new_points = jnp.max(_conv_mlp(new_points, params), axis=3)
    return new_xyz, new_points, idx


def kernel(point_cloud, W1a, b1a, W1b, b1b, W2a, b2a, W2b, b2b, W3a, b3a,
           W3b, b3b, Wf1, bf1, gf1, bef1, Wf2, bf2, gf2, bef2):
    l0_xyz = point_cloud
    l0_points = point_cloud
    l1_xyz, l1_points, _ = _sa_knn(l0_xyz, l0_points, 512, 12, [(W1a, b1a), (W1b, b1b)])
    l21_xyz, l21_points, _ = _sa_knn(l1_xyz, l1_points, 128, 8, [(W2a, b2a), (W2b, b2b)])
    new_points = jnp.concatenate([l21_xyz, l21_points], axis=1)[:, :, None, :]
    l31_points = jnp.max(_conv_mlp(new_points, [(W3a, b3a), (W3b, b3b)]), axis=3)
    x = jnp.squeeze(l31_points, axis=2)
    x = x @ Wf1.T + bf1
    x = gf1 * x / jnp.sqrt(1.0 + EPS) + bef1
    x = jax.nn.relu(x)
    x = x @ Wf2.T + bf2
    x = gf2 * x / jnp.sqrt(1.0 + EPS) + bef2
    x = jax.nn.relu(x)
    return (x, l21_points)


# speedup vs baseline: 33.0075x; 26.0095x over previous
import functools

import jax
import jax.numpy as jnp
from jax.experimental import pallas as pl

EPS = 1e-5


def _fps_kernel(pc_ref, fidx_ref, newxyz_ref, *, npoint, n):
    b = pc_ref.shape[0]
    X = pc_ref[:, 0, :]
    Y = pc_ref[:, 1, :]
    Z = pc_ref[:, 2, :]
    lane = jax.lax.broadcasted_iota(jnp.int32, (b, n), 1)
    subl = jax.lax.broadcasted_iota(jnp.int32, (b, n), 0)
    subl1 = jax.lax.broadcasted_iota(jnp.int32, (b, 1), 0)
    sublp = jax.lax.broadcasted_iota(jnp.int32, (b, npoint), 0)
    col = jax.lax.broadcasted_iota(jnp.int32, (b, npoint), 1)

    def body(i, carry):
        dists, far, aidx, ax, ay, az = carry
        sel = col == i
        aidx = jnp.where(sel, jnp.broadcast_to(far, aidx.shape), aidx)
        mask = lane == far
        cx = jnp.sum(jnp.where(mask, X, 0.0), axis=1, keepdims=True)
        cy = jnp.sum(jnp.where(mask, Y, 0.0), axis=1, keepdims=True)
        cz = jnp.sum(jnp.where(mask, Z, 0.0), axis=1, keepdims=True)
        ax = jnp.where(sel, jnp.broadcast_to(cx, ax.shape), ax)
        ay = jnp.where(sel, jnp.broadcast_to(cy, ay.shape), ay)
        az = jnp.where(sel, jnp.broadcast_to(cz, az.shape), az)
        d = (X - cx) ** 2 + (Y - cy) ** 2 + (Z - cz) ** 2
        dists = jnp.minimum(dists, d)
        m = jnp.max(dists, axis=1, keepdims=True)
        far = jnp.min(jnp.where(dists == m, lane, n), axis=1, keepdims=True)
        return dists, far, aidx, ax, ay, az

    dists0 = jnp.maximum(subl.astype(jnp.float32), 1e10)
    far0 = jnp.minimum(subl1, 0)
    aidx0 = jnp.minimum(sublp, 0)
    az0 = aidx0.astype(jnp.float32)
    _, _, aidx, ax, ay, az = jax.lax.fori_loop(
        0, npoint, body, (dists0, far0, aidx0, az0, az0, az0))
    fidx_ref[...] = aidx
    newxyz_ref[:, 0, :] = ax
    newxyz_ref[:, 1, :] = ay
    newxyz_ref[:, 2, :] = az


def _fps(pc, npoint):
    b, _, n = pc.shape
    return pl.pallas_call(
        functools.partial(_fps_kernel, npoint=npoint, n=n),
        out_shape=[jax.ShapeDtypeStruct((b, npoint), jnp.int32),
                   jax.ShapeDtypeStruct((b, 3, npoint), jnp.float32)],
    )(pc)


def _sq_dists(C, P):
    ones3 = jnp.zeros((3, 1), dtype=jnp.float32) + 1.0
    pn = jnp.sum(P * P, axis=0, keepdims=True)
    cn = jax.lax.dot_general(C * C, ones3, (((0,), (0,)), ((), ())))
    cp = jax.lax.dot_general(C, P, (((0,), (0,)), ((), ())))
    return (cn + pn) - 2.0 * cp


def _knn_mlp_pool(sq, F, beta, W2, k):
    s, n = sq.shape
    lane = jax.lax.broadcasted_iota(jnp.int32, (s, n), 1)
    acc = None
    for j in range(k):
        m = jnp.min(sq, axis=1, keepdims=True)
        eq = sq == m
        first = jnp.min(jnp.where(eq, lane, n), axis=1, keepdims=True)
        one = lane == first
        maskf = one.astype(jnp.float32)
        G = jax.lax.dot_general(maskf, F, (((1,), (1,)), ((), ())))
        Hh = jax.nn.relu(G + beta)
        O = jax.lax.dot_general(Hh, W2, (((1,), (1,)), ((), ())))
        acc = O if acc is None else jnp.maximum(acc, O)
        if j + 1 < k:
            sq = jnp.where(one, jnp.inf, sq)
    return acc


def _sa1_kernel(pc_ref, cxyz_ref, Wp_ref, A1_ref, b1_ref, W2_ref, out_ref,
                *, k):
    P = pc_ref[0]
    C = cxyz_ref[0]
    sq = _sq_dists(C, P)
    F = jax.lax.dot_general(Wp_ref[...], P, (((1,), (0,)), ((), ())))
    beta = b1_ref[...] - jax.lax.dot_general(
        C, A1_ref[...], (((0,), (1,)), ((), ())))
    out_ref[0] = _knn_mlp_pool(sq, F, beta, W2_ref[...], k)


def _sa2_kernel(pc_ref, cxyz_ref, feat_ref, A1_ref, A2_ref, b1_ref, W2_ref,
                out_ref, *, k):
    P = pc_ref[0]
    C = cxyz_ref[0]
    feat = feat_ref[0]
    sq = _sq_dists(C, P)
    F = (jax.lax.dot_general(A1_ref[...], P, (((1,), (0,)), ((), ())))
         + jax.lax.dot_general(A2_ref[...], feat, (((1,), (1,)), ((), ()))))
    beta = b1_ref[...] - jax.lax.dot_general(
        C, A1_ref[...], (((0,), (1,)), ((), ())))
    out_ref[0] = _knn_mlp_pool(sq, F, beta, W2_ref[...], k)


def _sa1(pc, cxyz, Wp, A1, b1, W2, k):
    b, _, n = pc.shape
    s = cxyz.shape[2]
    h2 = W2.shape[0]
    grid = (b,)
    return pl.pallas_call(
        functools.partial(_sa1_kernel, k=k),
        grid=grid,
        in_specs=[
            pl.BlockSpec((1, 3, n), lambda i: (i, 0, 0)),
            pl.BlockSpec((1, 3, s), lambda i: (i, 0, 0)),
            pl.BlockSpec(Wp.shape, lambda i: (0, 0)),
            pl.BlockSpec(A1.shape, lambda i: (0, 0)),
            pl.BlockSpec(b1.shape, lambda i: (0, 0)),
            pl.BlockSpec(W2.shape, lambda i: (0, 0)),
        ],
        out_specs=pl.BlockSpec((1, s, h2), lambda i: (i, 0, 0)),
        out_shape=jax.ShapeDtypeStruct((b, s, h2), jnp.float32),
    )(pc, cxyz, Wp, A1, b1, W2)


def _sa2(pc, cxyz, feat, A1, A2, b1, W2, k):
    b, _, n = pc.shape
    s = cxyz.shape[2]
    cf = feat.shape[2]
    h2 = W2.shape[0]
    grid = (b,)
    return pl.pallas_call(
        functools.partial(_sa2_kernel, k=k),
        grid=grid,
        in_specs=[
            pl.BlockSpec((1, 3, n), lambda i: (i, 0, 0)),
            pl.BlockSpec((1, 3, s), lambda i: (i, 0, 0)),
            pl.BlockSpec((1, n, cf), lambda i: (i, 0, 0)),
            pl.BlockSpec(A1.shape, lambda i: (0, 0)),
            pl.BlockSpec(A2.shape, lambda i: (0, 0)),
            pl.BlockSpec(b1.shape, lambda i: (0, 0)),
            pl.BlockSpec(W2.shape, lambda i: (0, 0)),
        ],
        out_specs=pl.BlockSpec((1, s, h2), lambda i: (i, 0, 0)),
        out_shape=jax.ShapeDtypeStruct((b, s, h2), jnp.float32),
    )(pc, cxyz, feat, A1, A2, b1, W2)


def _tail_kernel(xyz_ref, feat_ref, W3x_ref, W3f_ref, b3a_ref, W3b_ref,
                 b3b_ref, Wf1_ref, bn1_ref, W1s_ref, Wf2_ref, bn2_ref,
                 W2s_ref, logit_ref):
    xyz = xyz_ref[0]
    feat = feat_ref[0]
    h = jax.nn.relu(
        jax.lax.dot_general(W3x_ref[...], xyz, (((1,), (0,)), ((), ())))
        + jax.lax.dot_general(W3f_ref[...], feat, (((1,), (1,)), ((), ())))
        + b3a_ref[...])
    o = jax.lax.dot_general(W3b_ref[...], h, (((1,), (0,)), ((), ())))
    o = o + b3b_ref[...]
    x = jnp.max(o, axis=1, keepdims=True)
    rs = jnp.sqrt(1.0 + EPS)
    y = jax.lax.dot_general(Wf1_ref[...], x, (((1,), (0,)), ((), ())))
    y = W1s_ref[...] * (y + bn1_ref[..., 0:1]) / rs + bn1_ref[..., 1:2]
    y = jax.nn.relu(y)
    z = jax.lax.dot_general(Wf2_ref[...], y, (((1,), (0,)), ((), ())))
    z = W2s_ref[...] * (z + bn2_ref[..., 0:1]) / rs + bn2_ref[..., 1:2]
    logit_ref[0] = jax.nn.relu(z)


def _tail(xyz, feat, W3a, b3a, W3b, b3b, Wf1, bf1, gf1, bef1, Wf2, bf2, gf2,
          bef2):
    b, _, s = xyz.shape
    W3x = W3a[:, :3]
    W3f = W3a[:, 3:]
    b3ac = b3a[:, None]
    b3bc = b3b[:, None]
    bn1 = jnp.stack([bf1, bef1], axis=1)
    bn2 = jnp.stack([bf2, bef2], axis=1)
    g1c = gf1[:, None]
    g2c = gf2[:, None]
    grid = (b,)
    full = lambda a: pl.BlockSpec(a.shape, lambda i: tuple(0 for _ in a.shape))
    out = pl.pallas_call(
        _tail_kernel,
        grid=grid,
        in_specs=[
            pl.BlockSpec((1, 3, s), lambda i: (i, 0, 0)),
            pl.BlockSpec((1, s, feat.shape[2]), lambda i: (i, 0, 0)),
            full(W3x), full(W3f), full(b3ac), full(W3b), full(b3bc),
            full(Wf1), full(bn1), full(g1c), full(Wf2), full(bn2), full(g2c),
        ],
        out_specs=pl.BlockSpec((1, 128, 1), lambda i: (i, 0, 0)),
        out_shape=jax.ShapeDtypeStruct((b, 128, 1), jnp.float32),
    )(xyz, feat, W3x, W3f, b3ac, W3b, b3bc, Wf1, bn1, g1c, Wf2, bn2, g2c)
    return out[:, :, 0]


def kernel(point_cloud, W1a, b1a, W1b, b1b, W2a, b2a, W2b, b2b, W3a, b3a,
           W3b, b3b, Wf1, bf1, gf1, bef1, Wf2, bf2, gf2, bef2):
    pc = point_cloud
    _, l1_xyz = _fps(pc, 512)
    Wp1 = W1a[:, :3] + W1a[:, 3:]
    A1_1 = W1a[:, :3]
    p1 = _sa1(pc, l1_xyz, Wp1, A1_1, b1a[None, :], W1b, 12)
    _, l2_xyz = _fps(l1_xyz, 128)
    A1_2 = W2a[:, :3]
    A2_2 = W2a[:, 3:]
    p2 = _sa2(l1_xyz, l2_xyz, p1, A1_2, A2_2, b2a[None, :], W2b, 8)
    logit = _tail(l2_xyz, p2, W3a, b3a, W3b, b3b,
                  Wf1, bf1, gf1, bef1, Wf2, bf2, gf2, bef2)
    l21_points = jnp.transpose(p2, (0, 2, 1))
    return (logit, l21_points)
